# baseline (speedup 1.0000x reference)
import functools

import jax
import jax.numpy as jnp
from jax import lax
from jax.experimental import pallas as pl
from jax.experimental.pallas import tpu as pltpu
from jax.experimental.pallas import tpu_sc as plsc

N = 10000
E = 320000
H = 128
B = 64
HP = 144
NC = 2
NS = 16
NW = NC * NS
EPT = E // NW
CK = 40
NCH = EPT // CK
RPT = N // NS
NSLOT = 8
NGB = 3
NSB = 2


def _tc_layer_body(has_prev, x_ref, acc_ref, aself_ref_in, b_ref,
                   W_ref, as_ref, ad_ref, We_ref, ae_ref, mew_ref,
                   xwp_ref, asrc_ref, adst_ref, aself_ref, c16_ref):
    if has_prev:
        es = jnp.exp(aself_ref_in[...])
        xw_prev = x_ref[...]
        num = es * xw_prev + acc_ref[0, :, :H] + acc_ref[1, :, :H]
        den = es + acc_ref[0, :, H:H + 1] + acc_ref[1, :, H:H + 1]
        h = num / den + b_ref[...]
        h = jnp.maximum(h, 0.0)
    else:
        h = x_ref[...]
    W = W_ref[...]
    xw = jnp.dot(h, W, preferred_element_type=jnp.float32)
    asrc = jnp.dot(xw, as_ref[...], preferred_element_type=jnp.float32)
    adst = jnp.dot(xw, ad_ref[...], preferred_element_type=jnp.float32)
    c = jnp.sum(We_ref[...] * ae_ref[...])
    mew = mew_ref[0, 0]
    t = asrc + adst + c * mew
    aself = jnp.where(t >= 0, t, 0.2 * t)
    xwp_ref[...] = xw
    asrc_ref[...] = asrc
    adst_ref[...] = adst
    aself_ref[...] = aself
    c16_ref[...] = jnp.full((1, 16), c, dtype=jnp.float32)


BN = 2000
NB = N // BN


def _tc_layer(x, acc, aself_prev, b2d, W, a_s2d, a_d2d, We, a_e2d, mew):
    has_prev = acc is not None
    out_shape = [
        jax.ShapeDtypeStruct((N, H), jnp.float32),
        jax.ShapeDtypeStruct((N, 1), jnp.float32),
        jax.ShapeDtypeStruct((N, 1), jnp.float32),
        jax.ShapeDtypeStruct((N, 1), jnp.float32),
        jax.ShapeDtypeStruct((1, 16), jnp.float32),
    ]
    row = lambda shp: pl.BlockSpec(shp, lambda i: (i, 0))
    const = lambda shp: pl.BlockSpec(shp, lambda i: (0, 0))
    out_specs = [row((BN, H)), row((BN, 1)), row((BN, 1)), row((BN, 1)),
                 const((1, 16))]
    wspecs = [const((H, H)), const((H, 1)), const((H, 1)), const((1, H)),
              const((1, H)), const((1, 1))]
    if has_prev:
        body = functools.partial(_tc_layer_body, True)
        args = (x, acc, aself_prev, b2d, W, a_s2d, a_d2d, We, a_e2d, mew)
        in_specs = [row((BN, H)),
                    pl.BlockSpec((2, BN, HP), lambda i: (0, i, 0)),
                    row((BN, 1)), const((1, H))] + wspecs
    else:
        def body(x_ref, W_ref, as_ref, ad_ref, We_ref, ae_ref, mew_ref,
                 *outs):
            _tc_layer_body(False, x_ref, None, None, None, W_ref, as_ref,
                           ad_ref, We_ref, ae_ref, mew_ref, *outs)
        args = (x, W, a_s2d, a_d2d, We, a_e2d, mew)
        in_specs = [row((BN, H))] + wspecs
    return pl.pallas_call(body, out_shape=out_shape, grid=(NB,),
                          in_specs=in_specs, out_specs=out_specs)(*args)


def _tc_mean_body(ew_ref, mew_ref):
    mew_ref[...] = jnp.sum(ew_ref[...], axis=(0, 1), keepdims=True) / E


def _tc_final_body(xwp_ref, acc_ref, aself_ref, b_ref, batch_ref,
                   linW_ref, linb_ref, out_ref):
    es = jnp.exp(aself_ref[...])
    xw = xwp_ref[...]
    num = es * xw + acc_ref[0, :, :H] + acc_ref[1, :, :H]
    den = es + acc_ref[0, :, H:H + 1] + acc_ref[1, :, H:H + 1]
    h = num / den + b_ref[...]
    seg = jax.lax.broadcasted_iota(jnp.int32, (B, N), 0)
    M = jnp.where(batch_ref[...] == seg, 1.0, 0.0)
    cnt = jnp.sum(M, axis=1, keepdims=True)
    pooled = jnp.dot(M, h, preferred_element_type=jnp.float32)
    pooled = pooled / jnp.maximum(cnt, 1.0)
    logit = jnp.dot(pooled, linW_ref[...], preferred_element_type=jnp.float32)
    out_ref[...] = jax.nn.sigmoid(logit + linb_ref[...])


def _sc_edge_body(xwp_hbm, asrc_hbm, adst_hbm, c16_hbm, e3_hbm,
                  acc_hbm,
                  asrc_v, adst_v, c_v, idxr, gb0, gb1, gb2, sb0, sb1, acc_sh,
                  sem_g0, sem_g1, sem_g2, sem_s0, sem_s1, isem):
    cid = lax.axis_index("c")
    sid = lax.axis_index("s")
    wid = sid * NC + cid

    pltpu.sync_copy(asrc_hbm, asrc_v)
    pltpu.sync_copy(adst_hbm, adst_v)
    pltpu.sync_copy(c16_hbm, c_v)

    def zero_row(r, _):
        for v in range(HP // 16):
            sb0[r, pl.ds(v * 16, 16)] = jnp.zeros((16,), jnp.float32)
        return ()
    lax.fori_loop(0, CK, zero_row, (), unroll=4)
    base = sid * RPT
    nz = RPT // CK
    rem = RPT % CK
    for k in range(nz):
        pltpu.async_copy(sb0, acc_sh.at[pl.ds(base + k * CK, CK)], sem_s0)
    if rem:
        pltpu.async_copy(sb0.at[pl.ds(0, rem)],
                         acc_sh.at[pl.ds(base + nz * CK, rem)], sem_s0)
    for k in range(nz):
        pltpu.make_async_copy(sb0, acc_sh.at[pl.ds(base + k * CK, CK)],
                              sem_s0).wait()
    if rem:
        pltpu.make_async_copy(sb0.at[pl.ds(0, rem)],
                              acc_sh.at[pl.ds(base + nz * CK, rem)],
                              sem_s0).wait()
    plsc.subcore_barrier()

    c_vec = c_v[...]
    gbufs = (gb0, gb1, gb2)
    sbufs = (sb0, sb1)
    gsems = (sem_g0, sem_g1, sem_g2)
    ssems = (sem_s0, sem_s1)

    def slot_of(ch):
        return lax.rem(ch, NSLOT)

    def stage_idx(ch):
        sl = slot_of(ch)
        pltpu.async_copy(e3_hbm.at[wid, ch], idxr.at[sl], isem.at[sl])

    def wait_idx(ch):
        sl = slot_of(ch)
        pltpu.make_async_copy(e3_hbm.at[wid, ch], idxr.at[sl],
                              isem.at[sl]).wait()

    def issue_gather(ch, g):
        sl = slot_of(ch)
        pltpu.async_copy(xwp_hbm.at[idxr.at[sl, 0]], gbufs[g], gsems[g])

    def wait_gather(ch, g):
        sl = slot_of(ch)
        pltpu.make_async_copy(xwp_hbm.at[idxr.at[sl, 0]], gbufs[g],
                              gsems[g]).wait()

    def issue_scatter(ch, s):
        sl = slot_of(ch)
        pltpu.async_copy(sbufs[s], acc_sh.at[idxr.at[sl, 1]], ssems[s],
                         add=True)

    def wait_scatter(ch, s):
        sl = slot_of(ch)
        pltpu.make_async_copy(sbufs[s], acc_sh.at[idxr.at[sl, 1]],
                              ssems[s]).wait()

    def do_chunk(ch, g, s, steady=True):
        wait_gather(ch, g)
        sl = slot_of(ch)

        if steady:
            wait_scatter(ch - NSB, s)
        elif isinstance(ch, int) and ch >= NSB:
            wait_scatter(ch - NSB, s)

        for off, j0 in ((0, 0), (16, 0), (24, 8)):
            s16 = idxr[sl, 0, pl.ds(off, 16)]
            d16 = idxr[sl, 1, pl.ds(off, 16)]
            ewbits = idxr[sl, 2, pl.ds(off, 16)]
            ew16 = plsc.bitcast(ewbits, jnp.float32)
            asrc_g = plsc.load_gather(asrc_v, [s16])
            adst_g = plsc.load_gather(adst_v, [d16])
            t = asrc_g + adst_g + c_vec * ew16
            alpha = jnp.where(t >= 0, t, 0.2 * t)
            w16 = jnp.exp(alpha)
            for j in range(j0, 16):
                r = off + j
                wspl = jnp.full((16,), w16[j])
                for v in range(H // 32):
                    x32 = gbufs[g][r, pl.ds(v * 32, 32)]
                    a, b2 = plsc.unpack(
                        x32, format=plsc.PackFormat.INTERLEAVED,
                        preferred_element_type=jnp.float32)
                    sbufs[s][r, pl.ds(v * 32, 16)] = a * wspl
                    sbufs[s][r, pl.ds(v * 32 + 16, 16)] = b2 * wspl
                sbufs[s][r, pl.ds(H, 16)] = wspl

        issue_scatter(ch, s)

        if steady:
            wait_idx(ch + NGB)
            issue_gather(ch + NGB, g)
            stage_idx(ch + NGB + 2)
        else:
            if not isinstance(ch, int) or ch + NGB < NCH:
                wait_idx(ch + NGB)
                issue_gather(ch + NGB, g)
            if not isinstance(ch, int) or ch + NGB + 2 < NCH:
                stage_idx(ch + NGB + 2)

    for ch in range(NGB + 2):
        stage_idx(ch)
    for ch in range(NGB):
        wait_idx(ch)
        issue_gather(ch, ch)

    STEP = NGB * NSB
    head = NSB
    tail_start = NCH - NGB - 2
    while (tail_start - head) % STEP:
        tail_start -= 1
    for ch in range(head):
        do_chunk(ch, ch % NGB, ch % NSB, steady=False)

    def p2(i, _):
        for k in range(STEP):
            ch = head + i * STEP + k
            do_chunk(ch, (head + k) % NGB, (head + k) % NSB)
        return ()
    lax.fori_loop(0, (tail_start - head) // STEP, p2, ())
    for ch in range(tail_start, NCH):
        do_chunk(ch, ch % NGB, ch % NSB, steady=False)

    for s in range(NSB):
        ch = NCH - NSB + s
        wait_scatter(ch, ch % NSB)

    plsc.subcore_barrier()

    pltpu.sync_copy(acc_sh.at[pl.ds(base, RPT)],
                    acc_hbm.at[cid, pl.ds(base, RPT)])


def _sc_edge_call(xwp, asrc, adst, c16, e3):
    mesh = plsc.VectorSubcoreMesh(core_axis_name="c", subcore_axis_name="s",
                                  num_cores=NC, num_subcores=NS)
    f32 = jnp.float32
    kern = pl.kernel(
        _sc_edge_body,
        out_type=jax.ShapeDtypeStruct((NC, N, HP), f32),
        mesh=mesh,
        compiler_params=pltpu.CompilerParams(use_tc_tiling_on_sc=False,
                                             needs_layout_passes=False),
        scratch_types=[
            pltpu.VMEM((N,), f32),
            pltpu.VMEM((N,), f32),
            pltpu.VMEM((16,), f32),
            pltpu.VMEM((NSLOT, 3, CK), jnp.int32),
            pltpu.VMEM((CK, H), jnp.bfloat16),
            pltpu.VMEM((CK, H), jnp.bfloat16),
            pltpu.VMEM((CK, H), jnp.bfloat16),
            pltpu.VMEM((CK, HP), f32),
            pltpu.VMEM((CK, HP), f32),
            pltpu.VMEM_SHARED((N, HP), f32),
            pltpu.SemaphoreType.DMA,
            pltpu.SemaphoreType.DMA,
            pltpu.SemaphoreType.DMA,
            pltpu.SemaphoreType.DMA,
            pltpu.SemaphoreType.DMA,
            pltpu.SemaphoreType.DMA((NSLOT,)),
        ],
    )
    return kern(xwp, asrc, adst, c16, e3)


def kernel(x, edge_index, edge_weight, batch,
           W1, as1, ad1, We1, ae1, b1,
           W2, as2, ad2, We2, ae2, b2,
           W3, as3, ad3, We3, ae3, b3,
           linW, linb):
    f32 = jnp.float32
    src = edge_index[0]
    dst = edge_index[1]
    ew_bits = lax.bitcast_convert_type(edge_weight[:, 0], jnp.int32)
    e3 = jnp.stack([src.reshape(NW, NCH, CK),
                    dst.reshape(NW, NCH, CK),
                    ew_bits.reshape(NW, NCH, CK)], axis=2)

    mew = pl.pallas_call(
        _tc_mean_body,
        out_shape=jax.ShapeDtypeStruct((1, 1), f32),
    )(edge_weight.reshape(E // H, H))

    layers = [
        (W1, as1, ad1, We1, ae1, None),
        (W2, as2, ad2, We2, ae2, b1),
        (W3, as3, ad3, We3, ae3, b2),
    ]

    xwp = None
    acc = None
    aself = None
    for (W, a_s, a_d, We, a_e, b_prev) in layers:
        xwp, asrc, adst, aself, c16 = _tc_layer(
            xwp if acc is not None else x,
            acc, aself,
            b_prev.reshape(1, H) if b_prev is not None else None,
            W, a_s.reshape(H, 1), a_d.reshape(H, 1),
            We, a_e.reshape(1, H), mew)
        xwb = (xwp.reshape(N, H // 32, 2, 16).swapaxes(2, 3)
               .reshape(N, H).astype(jnp.bfloat16))
        acc = _sc_edge_call(xwb, asrc.reshape(N), adst.reshape(N),
                            c16.reshape(16), e3)

    out = pl.pallas_call(
        _tc_final_body,
        out_shape=jax.ShapeDtypeStruct((B, 1), f32),
    )(xwp, acc, aself, b3.reshape(1, H), batch.reshape(1, N), linW,
      linb.reshape(1, 1))
    return out


# speedup vs baseline: 1.0835x; 1.0835x over previous
import functools

import jax
import jax.numpy as jnp
from jax import lax
from jax.experimental import pallas as pl
from jax.experimental.pallas import tpu as pltpu
from jax.experimental.pallas import tpu_sc as plsc

N = 10000
E = 320000
H = 128
B = 64
HP = 136
NC = 2
NS = 16
NW = NC * NS
EPT = E // NW
CK = 40
NCH = EPT // CK
RPT = N // NS
NSLOT = 8
NGB = 4
NSB = 2


def _tc_layer_body(has_prev, x_ref, acc_ref, aself_ref_in, b_ref,
                   W_ref, as_ref, ad_ref, We_ref, ae_ref, mew_ref,
                   xwp_ref, adst_ref, aself_ref, c16_ref):
    if has_prev:
        es = jnp.exp(aself_ref_in[...])
        xw_prev = x_ref[:, :H]
        num = es * xw_prev + acc_ref[0, :, :H] + acc_ref[1, :, :H]
        den = es + acc_ref[0, :, H:H + 1] + acc_ref[1, :, H:H + 1]
        h = num / den + b_ref[...]
        h = jnp.maximum(h, 0.0)
    else:
        h = x_ref[...]
    W = W_ref[...]
    xw = jnp.dot(h, W, preferred_element_type=jnp.float32)
    asrc = jnp.dot(xw, as_ref[...], preferred_element_type=jnp.float32)
    adst = jnp.dot(xw, ad_ref[...], preferred_element_type=jnp.float32)
    c = jnp.sum(We_ref[...] * ae_ref[...])
    mew = mew_ref[0, 0]
    t = asrc + adst + c * mew
    aself = jnp.where(t >= 0, t, 0.2 * t)
    rows = xw.shape[0]
    ones = jnp.ones((rows, 1), dtype=jnp.float32)
    pad = jnp.zeros((rows, HP - H - 2), dtype=jnp.float32)
    xwp_ref[...] = jnp.concatenate([xw, ones, asrc, pad], axis=1)
    adst_ref[...] = adst
    aself_ref[...] = aself
    c16_ref[...] = jnp.full((1, 16), c, dtype=jnp.float32)


BN = 2000
NB = N // BN


def _tc_layer(x, acc, aself_prev, b2d, W, a_s2d, a_d2d, We, a_e2d, mew):
    has_prev = acc is not None
    out_shape = [
        jax.ShapeDtypeStruct((N, HP), jnp.float32),
        jax.ShapeDtypeStruct((N, 1), jnp.float32),
        jax.ShapeDtypeStruct((N, 1), jnp.float32),
        jax.ShapeDtypeStruct((1, 16), jnp.float32),
    ]
    row = lambda shp: pl.BlockSpec(shp, lambda i: (i, 0))
    const = lambda shp: pl.BlockSpec(shp, lambda i: (0, 0))
    out_specs = [row((BN, HP)), row((BN, 1)), row((BN, 1)), const((1, 16))]
    wspecs = [const((H, H)), const((H, 1)), const((H, 1)), const((1, H)),
              const((1, H)), const((1, 1))]
    if has_prev:
        body = functools.partial(_tc_layer_body, True)
        args = (x, acc, aself_prev, b2d, W, a_s2d, a_d2d, We, a_e2d, mew)
        in_specs = [row((BN, HP)),
                    pl.BlockSpec((2, BN, HP), lambda i: (0, i, 0)),
                    row((BN, 1)), const((1, H))] + wspecs
    else:
        def body(x_ref, W_ref, as_ref, ad_ref, We_ref, ae_ref, mew_ref,
                 *outs):
            _tc_layer_body(False, x_ref, None, None, None, W_ref, as_ref,
                           ad_ref, We_ref, ae_ref, mew_ref, *outs)
        args = (x, W, a_s2d, a_d2d, We, a_e2d, mew)
        in_specs = [row((BN, H))] + wspecs
    return pl.pallas_call(body, out_shape=out_shape, grid=(NB,),
                          in_specs=in_specs, out_specs=out_specs)(*args)


def _tc_mean_body(ew_ref, mew_ref):
    mew_ref[...] = jnp.sum(ew_ref[...], axis=(0, 1), keepdims=True) / E


def _tc_final_body(xwp_ref, acc_ref, aself_ref, b_ref, batch_ref,
                   linW_ref, linb_ref, out_ref):
    es = jnp.exp(aself_ref[...])
    xw = xwp_ref[:, :H]
    num = es * xw + acc_ref[0, :, :H] + acc_ref[1, :, :H]
    den = es + acc_ref[0, :, H:H + 1] + acc_ref[1, :, H:H + 1]
    h = num / den + b_ref[...]
    seg = jax.lax.broadcasted_iota(jnp.int32, (B, N), 0)
    M = jnp.where(batch_ref[...] == seg, 1.0, 0.0)
    cnt = jnp.sum(M, axis=1, keepdims=True)
    pooled = jnp.dot(M, h, preferred_element_type=jnp.float32)
    pooled = pooled / jnp.maximum(cnt, 1.0)
    logit = jnp.dot(pooled, linW_ref[...], preferred_element_type=jnp.float32)
    out_ref[...] = jax.nn.sigmoid(logit + linb_ref[...])


def _sc_edge_body(xwp_hbm, adst_hbm, c16_hbm, e3_hbm,
                  acc_hbm,
                  adst_v, c_v, idxr, gb0, gb1, gb2, gb3, sb0, sb1, acc_sh,
                  sem_g0, sem_g1, sem_g2, sem_g3, sem_s0, sem_s1, isem):
    cid = lax.axis_index("c")
    sid = lax.axis_index("s")
    wid = sid * NC + cid

    pltpu.sync_copy(adst_hbm, adst_v)
    pltpu.sync_copy(c16_hbm, c_v)

    def zero_row(r, _):
        for off in list(range(0, H, 16)) + [HP - 16]:
            sb0[r, pl.ds(off, 16)] = jnp.zeros((16,), jnp.float32)
        return ()
    lax.fori_loop(0, CK, zero_row, (), unroll=4)
    base = sid * RPT
    nz = RPT // CK
    rem = RPT % CK
    for k in range(nz):
        pltpu.async_copy(sb0, acc_sh.at[pl.ds(base + k * CK, CK)], sem_s0)
    if rem:
        pltpu.async_copy(sb0.at[pl.ds(0, rem)],
                         acc_sh.at[pl.ds(base + nz * CK, rem)], sem_s0)
    for k in range(nz):
        pltpu.make_async_copy(sb0, acc_sh.at[pl.ds(base + k * CK, CK)],
                              sem_s0).wait()
    if rem:
        pltpu.make_async_copy(sb0.at[pl.ds(0, rem)],
                              acc_sh.at[pl.ds(base + nz * CK, rem)],
                              sem_s0).wait()
    plsc.subcore_barrier()

    c_vec = c_v[...]
    gbufs = (gb0, gb1, gb2, gb3)
    sbufs = (sb0, sb1)
    gsems = (sem_g0, sem_g1, sem_g2, sem_g3)
    ssems = (sem_s0, sem_s1)

    def slot_of(ch):
        return lax.rem(ch, NSLOT)

    def stage_idx(ch):
        sl = slot_of(ch)
        pltpu.async_copy(e3_hbm.at[wid, ch], idxr.at[sl], isem.at[sl])

    def wait_idx(ch):
        sl = slot_of(ch)
        pltpu.make_async_copy(e3_hbm.at[wid, ch], idxr.at[sl],
                              isem.at[sl]).wait()

    def issue_gather(ch, g):
        sl = slot_of(ch)
        pltpu.async_copy(xwp_hbm.at[idxr.at[sl, 0]], gbufs[g], gsems[g])

    def wait_gather(ch, g):
        sl = slot_of(ch)
        pltpu.make_async_copy(xwp_hbm.at[idxr.at[sl, 0]], gbufs[g],
                              gsems[g]).wait()

    def issue_scatter(ch, s):
        sl = slot_of(ch)
        pltpu.async_copy(sbufs[s], acc_sh.at[idxr.at[sl, 1]], ssems[s],
                         add=True)

    def wait_scatter(ch, s):
        sl = slot_of(ch)
        pltpu.make_async_copy(sbufs[s], acc_sh.at[idxr.at[sl, 1]],
                              ssems[s]).wait()

    def do_chunk(ch, g, s, steady=True):
        wait_gather(ch, g)
        sl = slot_of(ch)

        if steady:
            wait_scatter(ch - NSB, s)
        elif isinstance(ch, int) and ch >= NSB:
            wait_scatter(ch - NSB, s)

        col129 = jnp.full((16,), H + 1, jnp.int32)
        for off, j0 in ((0, 0), (16, 0), (24, 8)):
            rows16 = lax.iota(jnp.int32, 16) + off
            asrc_g = plsc.load_gather(gbufs[g], [rows16, col129])
            d16 = idxr[sl, 1, pl.ds(off, 16)]
            ewbits = idxr[sl, 2, pl.ds(off, 16)]
            ew16 = plsc.bitcast(ewbits, jnp.float32)
            adst_g = plsc.load_gather(adst_v, [d16])
            t = asrc_g + adst_g + c_vec * ew16
            alpha = jnp.where(t >= 0, t, 0.2 * t)
            w16 = jnp.exp(alpha)
            for j in range(j0, 16):
                r = off + j
                wspl = jnp.full((16,), w16[j])
                sbufs[s][r, pl.ds(HP - 16, 16)] = wspl
                for v in range(H // 16):
                    cs = pl.ds(v * 16, 16)
                    sbufs[s][r, cs] = gbufs[g][r, cs] * wspl

        issue_scatter(ch, s)

        if steady:
            wait_idx(ch + NGB)
            issue_gather(ch + NGB, g)
            stage_idx(ch + NGB + 2)
        else:
            if not isinstance(ch, int) or ch + NGB < NCH:
                wait_idx(ch + NGB)
                issue_gather(ch + NGB, g)
            if not isinstance(ch, int) or ch + NGB + 2 < NCH:
                stage_idx(ch + NGB + 2)

    for ch in range(NGB + 2):
        stage_idx(ch)
    for ch in range(NGB):
        wait_idx(ch)
        issue_gather(ch, ch)

    STEP = NGB
    head = NSB
    tail_start = NCH - NGB - 2
    while (tail_start - head) % STEP:
        tail_start -= 1
    for ch in range(head):
        do_chunk(ch, ch % NGB, ch % NSB, steady=False)

    def p2(i, _):
        for k in range(STEP):
            ch = head + i * STEP + k
            do_chunk(ch, (head + k) % NGB, (head + k) % NSB)
        return ()
    lax.fori_loop(0, (tail_start - head) // STEP, p2, ())
    for ch in range(tail_start, NCH):
        do_chunk(ch, ch % NGB, ch % NSB, steady=False)

    for s in range(NSB):
        ch = NCH - NSB + s
        wait_scatter(ch, ch % NSB)

    plsc.subcore_barrier()

    pltpu.sync_copy(acc_sh.at[pl.ds(base, RPT)],
                    acc_hbm.at[cid, pl.ds(base, RPT)])


def _sc_edge_call(xwp, adst, c16, e3):
    mesh = plsc.VectorSubcoreMesh(core_axis_name="c", subcore_axis_name="s",
                                  num_cores=NC, num_subcores=NS)
    f32 = jnp.float32
    kern = pl.kernel(
        _sc_edge_body,
        out_type=jax.ShapeDtypeStruct((NC, N, HP), f32),
        mesh=mesh,
        compiler_params=pltpu.CompilerParams(use_tc_tiling_on_sc=False,
                                             needs_layout_passes=False),
        scratch_types=[
            pltpu.VMEM((N,), f32),
            pltpu.VMEM((16,), f32),
            pltpu.VMEM((NSLOT, 3, CK), jnp.int32),
            pltpu.VMEM((CK, HP), f32),
            pltpu.VMEM((CK, HP), f32),
            pltpu.VMEM((CK, HP), f32),
            pltpu.VMEM((CK, HP), f32),
            pltpu.VMEM((CK, HP), f32),
            pltpu.VMEM((CK, HP), f32),
            pltpu.VMEM_SHARED((N, HP), f32),
            pltpu.SemaphoreType.DMA,
            pltpu.SemaphoreType.DMA,
            pltpu.SemaphoreType.DMA,
            pltpu.SemaphoreType.DMA,
            pltpu.SemaphoreType.DMA,
            pltpu.SemaphoreType.DMA,
            pltpu.SemaphoreType.DMA((NSLOT,)),
        ],
    )
    return kern(xwp, adst, c16, e3)


def kernel(x, edge_index, edge_weight, batch,
           W1, as1, ad1, We1, ae1, b1,
           W2, as2, ad2, We2, ae2, b2,
           W3, as3, ad3, We3, ae3, b3,
           linW, linb):
    f32 = jnp.float32
    src = edge_index[0]
    dst = edge_index[1]
    ew_bits = lax.bitcast_convert_type(edge_weight[:, 0], jnp.int32)
    e3 = jnp.stack([src.reshape(NW, NCH, CK),
                    dst.reshape(NW, NCH, CK),
                    ew_bits.reshape(NW, NCH, CK)], axis=2)

    mew = pl.pallas_call(
        _tc_mean_body,
        out_shape=jax.ShapeDtypeStruct((1, 1), f32),
    )(edge_weight.reshape(E // H, H))

    layers = [
        (W1, as1, ad1, We1, ae1, None),
        (W2, as2, ad2, We2, ae2, b1),
        (W3, as3, ad3, We3, ae3, b2),
    ]

    xwp = None
    acc = None
    aself = None
    for (W, a_s, a_d, We, a_e, b_prev) in layers:
        xwp, adst, aself, c16 = _tc_layer(
            xwp if acc is not None else x,
            acc, aself,
            b_prev.reshape(1, H) if b_prev is not None else None,
            W, a_s.reshape(H, 1), a_d.reshape(H, 1),
            We, a_e.reshape(1, H), mew)
        acc = _sc_edge_call(xwp, adst.reshape(N), c16.reshape(16), e3)

    out = pl.pallas_call(
        _tc_final_body,
        out_shape=jax.ShapeDtypeStruct((B, 1), f32),
    )(xwp, acc, aself, b3.reshape(1, H), batch.reshape(1, N), linW,
      linb.reshape(1, 1))
    return out


# speedup vs baseline: 1.0851x; 1.0015x over previous
import functools

import jax
import jax.numpy as jnp
from jax import lax
from jax.experimental import pallas as pl
from jax.experimental.pallas import tpu as pltpu
from jax.experimental.pallas import tpu_sc as plsc

N = 10000
E = 320000
H = 128
B = 64
HP = 136
NC = 2
NS = 16
NW = NC * NS
EPT = E // NW
CK = 40
NCH = EPT // CK
RPT = N // NS
NSLOT = 8
NGB = 4
NSB = 2


def _tc_layer_body(has_prev, mew, x_ref, acc_ref, aself_ref_in, b_ref,
                   W_ref, as_ref, ad_ref, We_ref, ae_ref,
                   xwp_ref, adst_ref, aself_ref, c16_ref):
    if has_prev:
        es = jnp.exp(aself_ref_in[...])
        xw_prev = x_ref[:, :H]
        num = es * xw_prev + acc_ref[0, :, :H] + acc_ref[1, :, :H]
        den = es + acc_ref[0, :, H:H + 1] + acc_ref[1, :, H:H + 1]
        h = num / den + b_ref[...]
        h = jnp.maximum(h, 0.0)
    else:
        h = x_ref[...]
    W = W_ref[...]
    xw = jnp.dot(h, W, preferred_element_type=jnp.float32)
    asrc = jnp.dot(xw, as_ref[...], preferred_element_type=jnp.float32)
    adst = jnp.dot(xw, ad_ref[...], preferred_element_type=jnp.float32)
    c = jnp.sum(We_ref[...] * ae_ref[...])
    t = asrc + adst + c * mew
    aself = jnp.where(t >= 0, t, 0.2 * t)
    rows = xw.shape[0]
    ones = jnp.ones((rows, 1), dtype=jnp.float32)
    pad = jnp.zeros((rows, HP - H - 2), dtype=jnp.float32)
    xwp_ref[...] = jnp.concatenate([xw, ones, asrc, pad], axis=1)
    adst_ref[...] = adst
    aself_ref[...] = aself
    c16_ref[...] = jnp.full((1, 16), c, dtype=jnp.float32)


BN = 2000
NB = N // BN


def _tc_layer(x, acc, aself_prev, b2d, W, a_s2d, a_d2d, We, a_e2d, mew):
    has_prev = acc is not None
    out_shape = [
        jax.ShapeDtypeStruct((N, HP), jnp.float32),
        jax.ShapeDtypeStruct((N, 1), jnp.float32),
        jax.ShapeDtypeStruct((N, 1), jnp.float32),
        jax.ShapeDtypeStruct((1, 16), jnp.float32),
    ]
    row = lambda shp: pl.BlockSpec(shp, lambda i: (i, 0))
    const = lambda shp: pl.BlockSpec(shp, lambda i: (0, 0))
    out_specs = [row((BN, HP)), row((BN, 1)), row((BN, 1)), const((1, 16))]
    wspecs = [const((H, H)), const((H, 1)), const((H, 1)), const((1, H)),
              const((1, H)), const((1, 1))]
    if has_prev:
        def body(x_ref, acc_ref, aself_in, b_ref, W_ref, as_ref, ad_ref,
                 We_ref, ae_ref, mew_ref, *outs):
            _tc_layer_body(True, mew_ref[0, 0], x_ref, acc_ref, aself_in,
                           b_ref, W_ref, as_ref, ad_ref, We_ref, ae_ref,
                           *outs)
        args = (x, acc, aself_prev, b2d, W, a_s2d, a_d2d, We, a_e2d, mew)
        in_specs = [row((BN, HP)),
                    pl.BlockSpec((2, BN, HP), lambda i: (0, i, 0)),
                    row((BN, 1)), const((1, H))] + wspecs[:-1] + [const((1, 1))]
        return pl.pallas_call(body, out_shape=out_shape, grid=(NB,),
                              in_specs=in_specs, out_specs=out_specs)(*args)
    else:
        def body(x_ref, W_ref, as_ref, ad_ref, We_ref, ae_ref, ew_ref,
                 xwp_ref, adst2_ref, aself2_ref, c2_ref, mew_ref):
            mval = jnp.sum(ew_ref[...]) / E
            mew_ref[...] = jnp.full((1, 1), mval, dtype=jnp.float32)
            _tc_layer_body(False, mval, x_ref, None, None, None, W_ref,
                           as_ref, ad_ref, We_ref, ae_ref,
                           xwp_ref, adst2_ref, aself2_ref, c2_ref)
        args = (x, W, a_s2d, a_d2d, We, a_e2d, mew)
        in_specs = ([row((BN, H))] + wspecs[:-1]
                    + [const((E // H, H))])
        out_shape2 = out_shape + [jax.ShapeDtypeStruct((1, 1), jnp.float32)]
        out_specs2 = out_specs + [const((1, 1))]
        return pl.pallas_call(body, out_shape=out_shape2, grid=(NB,),
                              in_specs=in_specs, out_specs=out_specs2)(*args)


def _tc_mean_body(ew_ref, mew_ref):
    mew_ref[...] = jnp.sum(ew_ref[...], axis=(0, 1), keepdims=True) / E


def _tc_final_body(xwp_ref, acc_ref, aself_ref, b_ref, batch_ref,
                   linW_ref, linb_ref, out_ref):
    es = jnp.exp(aself_ref[...])
    xw = xwp_ref[:, :H]
    num = es * xw + acc_ref[0, :, :H] + acc_ref[1, :, :H]
    den = es + acc_ref[0, :, H:H + 1] + acc_ref[1, :, H:H + 1]
    h = num / den + b_ref[...]
    seg = jax.lax.broadcasted_iota(jnp.int32, (B, N), 0)
    M = jnp.where(batch_ref[...] == seg, 1.0, 0.0)
    cnt = jnp.sum(M, axis=1, keepdims=True)
    pooled = jnp.dot(M, h, preferred_element_type=jnp.float32)
    pooled = pooled / jnp.maximum(cnt, 1.0)
    logit = jnp.dot(pooled, linW_ref[...], preferred_element_type=jnp.float32)
    out_ref[...] = jax.nn.sigmoid(logit + linb_ref[...])


def _sc_edge_body(xwp_hbm, adst_hbm, c16_hbm, e3_hbm,
                  acc_hbm,
                  adst_v, c_v, idxr, gb0, gb1, gb2, gb3, sb0, sb1, acc_sh,
                  sem_g0, sem_g1, sem_g2, sem_g3, sem_s0, sem_s1, isem):
    cid = lax.axis_index("c")
    sid = lax.axis_index("s")
    wid = sid * NC + cid

    pltpu.sync_copy(adst_hbm, adst_v)
    pltpu.sync_copy(c16_hbm, c_v)

    def zero_row(r, _):
        for off in list(range(0, H, 16)) + [HP - 16]:
            sb0[r, pl.ds(off, 16)] = jnp.zeros((16,), jnp.float32)
        return ()
    lax.fori_loop(0, CK, zero_row, (), unroll=4)
    base = sid * RPT
    nz = RPT // CK
    rem = RPT % CK
    for k in range(nz):
        pltpu.async_copy(sb0, acc_sh.at[pl.ds(base + k * CK, CK)], sem_s0)
    if rem:
        pltpu.async_copy(sb0.at[pl.ds(0, rem)],
                         acc_sh.at[pl.ds(base + nz * CK, rem)], sem_s0)
    for k in range(nz):
        pltpu.make_async_copy(sb0, acc_sh.at[pl.ds(base + k * CK, CK)],
                              sem_s0).wait()
    if rem:
        pltpu.make_async_copy(sb0.at[pl.ds(0, rem)],
                              acc_sh.at[pl.ds(base + nz * CK, rem)],
                              sem_s0).wait()
    plsc.subcore_barrier()

    c_vec = c_v[...]
    gbufs = (gb0, gb1, gb2, gb3)
    sbufs = (sb0, sb1)
    gsems = (sem_g0, sem_g1, sem_g2, sem_g3)
    ssems = (sem_s0, sem_s1)

    def slot_of(ch):
        return lax.rem(ch, NSLOT)

    def stage_idx(ch):
        sl = slot_of(ch)
        pltpu.async_copy(e3_hbm.at[wid, ch], idxr.at[sl], isem.at[sl])

    def wait_idx(ch):
        sl = slot_of(ch)
        pltpu.make_async_copy(e3_hbm.at[wid, ch], idxr.at[sl],
                              isem.at[sl]).wait()

    def issue_gather(ch, g):
        sl = slot_of(ch)
        pltpu.async_copy(xwp_hbm.at[idxr.at[sl, 0]], gbufs[g], gsems[g])

    def wait_gather(ch, g):
        sl = slot_of(ch)
        pltpu.make_async_copy(xwp_hbm.at[idxr.at[sl, 0]], gbufs[g],
                              gsems[g]).wait()

    def issue_scatter(ch, s):
        sl = slot_of(ch)
        pltpu.async_copy(sbufs[s], acc_sh.at[idxr.at[sl, 1]], ssems[s],
                         add=True)

    def wait_scatter(ch, s):
        sl = slot_of(ch)
        pltpu.make_async_copy(sbufs[s], acc_sh.at[idxr.at[sl, 1]],
                              ssems[s]).wait()

    def do_chunk(ch, g, s, steady=True):
        wait_gather(ch, g)
        sl = slot_of(ch)

        if steady:
            wait_scatter(ch - NSB, s)
        elif isinstance(ch, int) and ch >= NSB:
            wait_scatter(ch - NSB, s)

        col129 = jnp.full((16,), H + 1, jnp.int32)
        for off, j0 in ((0, 0), (16, 0), (24, 8)):
            rows16 = lax.iota(jnp.int32, 16) + off
            asrc_g = plsc.load_gather(gbufs[g], [rows16, col129])
            d16 = idxr[sl, 1, pl.ds(off, 16)]
            ewbits = idxr[sl, 2, pl.ds(off, 16)]
            ew16 = plsc.bitcast(ewbits, jnp.float32)
            adst_g = plsc.load_gather(adst_v, [d16])
            t = asrc_g + adst_g + c_vec * ew16
            alpha = jnp.where(t >= 0, t, 0.2 * t)
            w16 = jnp.exp(alpha)
            for j in range(j0, 16):
                r = off + j
                wspl = jnp.full((16,), w16[j])
                sbufs[s][r, pl.ds(HP - 16, 16)] = wspl
                for v in range(H // 16):
                    cs = pl.ds(v * 16, 16)
                    sbufs[s][r, cs] = gbufs[g][r, cs] * wspl

        issue_scatter(ch, s)

        if steady:
            wait_idx(ch + NGB)
            issue_gather(ch + NGB, g)
            stage_idx(ch + NGB + 2)
        else:
            if not isinstance(ch, int) or ch + NGB < NCH:
                wait_idx(ch + NGB)
                issue_gather(ch + NGB, g)
            if not isinstance(ch, int) or ch + NGB + 2 < NCH:
                stage_idx(ch + NGB + 2)

    for ch in range(NGB + 2):
        stage_idx(ch)
    for ch in range(NGB):
        wait_idx(ch)
        issue_gather(ch, ch)

    STEP = NGB
    head = NSB
    tail_start = NCH - NGB - 2
    while (tail_start - head) % STEP:
        tail_start -= 1
    for ch in range(head):
        do_chunk(ch, ch % NGB, ch % NSB, steady=False)

    def p2(i, _):
        for k in range(STEP):
            ch = head + i * STEP + k
            do_chunk(ch, (head + k) % NGB, (head + k) % NSB)
        return ()
    lax.fori_loop(0, (tail_start - head) // STEP, p2, ())
    for ch in range(tail_start, NCH):
        do_chunk(ch, ch % NGB, ch % NSB, steady=False)

    for s in range(NSB):
        ch = NCH - NSB + s
        wait_scatter(ch, ch % NSB)

    plsc.subcore_barrier()

    pltpu.sync_copy(acc_sh.at[pl.ds(base, RPT)],
                    acc_hbm.at[cid, pl.ds(base, RPT)])


def _sc_edge_call(xwp, adst, c16, e3):
    mesh = plsc.VectorSubcoreMesh(core_axis_name="c", subcore_axis_name="s",
                                  num_cores=NC, num_subcores=NS)
    f32 = jnp.float32
    kern = pl.kernel(
        _sc_edge_body,
        out_type=jax.ShapeDtypeStruct((NC, N, HP), f32),
        mesh=mesh,
        compiler_params=pltpu.CompilerParams(use_tc_tiling_on_sc=False,
                                             needs_layout_passes=False),
        scratch_types=[
            pltpu.VMEM((N,), f32),
            pltpu.VMEM((16,), f32),
            pltpu.VMEM((NSLOT, 3, CK), jnp.int32),
            pltpu.VMEM((CK, HP), f32),
            pltpu.VMEM((CK, HP), f32),
            pltpu.VMEM((CK, HP), f32),
            pltpu.VMEM((CK, HP), f32),
            pltpu.VMEM((CK, HP), f32),
            pltpu.VMEM((CK, HP), f32),
            pltpu.VMEM_SHARED((N, HP), f32),
            pltpu.SemaphoreType.DMA,
            pltpu.SemaphoreType.DMA,
            pltpu.SemaphoreType.DMA,
            pltpu.SemaphoreType.DMA,
            pltpu.SemaphoreType.DMA,
            pltpu.SemaphoreType.DMA,
            pltpu.SemaphoreType.DMA((NSLOT,)),
        ],
    )
    return kern(xwp, adst, c16, e3)


def kernel(x, edge_index, edge_weight, batch,
           W1, as1, ad1, We1, ae1, b1,
           W2, as2, ad2, We2, ae2, b2,
           W3, as3, ad3, We3, ae3, b3,
           linW, linb):
    f32 = jnp.float32
    src = edge_index[0]
    dst = edge_index[1]
    ew_bits = lax.bitcast_convert_type(edge_weight[:, 0], jnp.int32)
    e3 = jnp.stack([src.reshape(NW, NCH, CK),
                    dst.reshape(NW, NCH, CK),
                    ew_bits.reshape(NW, NCH, CK)], axis=2)

    ew2d = edge_weight.reshape(E // H, H)

    layers = [
        (W1, as1, ad1, We1, ae1, None),
        (W2, as2, ad2, We2, ae2, b1),
        (W3, as3, ad3, We3, ae3, b2),
    ]

    xwp = None
    acc = None
    aself = None
    mew = None
    for (W, a_s, a_d, We, a_e, b_prev) in layers:
        if acc is None:
            xwp, adst, aself, c16, mew = _tc_layer(
                x, None, None, None,
                W, a_s.reshape(H, 1), a_d.reshape(H, 1),
                We, a_e.reshape(1, H), ew2d)
        else:
            xwp, adst, aself, c16 = _tc_layer(
                xwp, acc, aself, b_prev.reshape(1, H),
                W, a_s.reshape(H, 1), a_d.reshape(H, 1),
                We, a_e.reshape(1, H), mew)
        acc = _sc_edge_call(xwp, adst.reshape(N), c16.reshape(16), e3)

    out = pl.pallas_call(
        _tc_final_body,
        out_shape=jax.ShapeDtypeStruct((B, 1), f32),
    )(xwp, acc, aself, b3.reshape(1, H), batch.reshape(1, N), linW,
      linb.reshape(1, 1))
    return out


# speedup vs baseline: 1.0883x; 1.0030x over previous
import functools

import jax
import jax.numpy as jnp
from jax import lax
from jax.experimental import pallas as pl
from jax.experimental.pallas import tpu as pltpu
from jax.experimental.pallas import tpu_sc as plsc

N = 10000
E = 320000
H = 128
B = 64
HP = 136
NC = 2
NS = 16
NW = NC * NS
EPT = E // NW
CK = 40
NCH = EPT // CK
RPT = N // NS
NSLOT = 8
NGB = 4
NSB = 2


def _tc_layer_body(has_prev, mew, x_ref, acc_ref, aself_ref_in, b_ref,
                   W_ref, as_ref, ad_ref, We_ref, ae_ref,
                   xwp_ref, adst_ref, aself_ref, c16_ref):
    if has_prev:
        es = jnp.exp(aself_ref_in[...])
        xw_prev = x_ref[:, :H]
        num = es * xw_prev + acc_ref[0, :, :H] + acc_ref[1, :, :H]
        den = es + acc_ref[0, :, H:H + 1] + acc_ref[1, :, H:H + 1]
        h = num / den + b_ref[...]
        h = jnp.maximum(h, 0.0)
    else:
        h = x_ref[...]
    W = W_ref[...]
    xw = jnp.dot(h, W, preferred_element_type=jnp.float32)
    asrc = jnp.dot(xw, as_ref[...], preferred_element_type=jnp.float32)
    adst = jnp.dot(xw, ad_ref[...], preferred_element_type=jnp.float32)
    c = jnp.sum(We_ref[...] * ae_ref[...])
    t = asrc + adst + c * mew
    aself = jnp.where(t >= 0, t, 0.2 * t)
    rows = xw.shape[0]
    ones = jnp.ones((rows, 1), dtype=jnp.float32)
    pad = jnp.zeros((rows, HP - H - 2), dtype=jnp.float32)
    xwp_ref[...] = jnp.concatenate([xw, ones, asrc, pad], axis=1)
    adst_ref[...] = adst
    aself_ref[...] = aself
    c16_ref[...] = jnp.full((1, 16), c, dtype=jnp.float32)


BN = 2000
NB = N // BN


def _tc_layer(x, acc, aself_prev, b2d, W, a_s2d, a_d2d, We, a_e2d, mew):
    has_prev = acc is not None
    out_shape = [
        jax.ShapeDtypeStruct((N, HP), jnp.float32),
        jax.ShapeDtypeStruct((N, 1), jnp.float32),
        jax.ShapeDtypeStruct((N, 1), jnp.float32),
        jax.ShapeDtypeStruct((1, 16), jnp.float32),
    ]
    row = lambda shp: pl.BlockSpec(shp, lambda i: (i, 0))
    const = lambda shp: pl.BlockSpec(shp, lambda i: (0, 0))
    out_specs = [row((BN, HP)), row((BN, 1)), row((BN, 1)), const((1, 16))]
    wspecs = [const((H, H)), const((H, 1)), const((H, 1)), const((1, H)),
              const((1, H)), const((1, 1))]
    if has_prev:
        def body(x_ref, acc_ref, aself_in, b_ref, W_ref, as_ref, ad_ref,
                 We_ref, ae_ref, mew_ref, *outs):
            _tc_layer_body(True, mew_ref[0, 0], x_ref, acc_ref, aself_in,
                           b_ref, W_ref, as_ref, ad_ref, We_ref, ae_ref,
                           *outs)
        args = (x, acc, aself_prev, b2d, W, a_s2d, a_d2d, We, a_e2d, mew)
        in_specs = [row((BN, HP)),
                    pl.BlockSpec((2, BN, HP), lambda i: (0, i, 0)),
                    row((BN, 1)), const((1, H))] + wspecs[:-1] + [const((1, 1))]
        return pl.pallas_call(body, out_shape=out_shape, grid=(NB,),
                              in_specs=in_specs, out_specs=out_specs)(*args)
    else:
        def body(x_ref, W_ref, as_ref, ad_ref, We_ref, ae_ref, ew_ref,
                 xwp_ref, adst2_ref, aself2_ref, c2_ref, mew_ref):
            mval = jnp.sum(ew_ref[...]) / E
            mew_ref[...] = jnp.full((1, 1), mval, dtype=jnp.float32)
            _tc_layer_body(False, mval, x_ref, None, None, None, W_ref,
                           as_ref, ad_ref, We_ref, ae_ref,
                           xwp_ref, adst2_ref, aself2_ref, c2_ref)
        args = (x, W, a_s2d, a_d2d, We, a_e2d, mew)
        in_specs = ([row((BN, H))] + wspecs[:-1]
                    + [const((E // H, H))])
        out_shape2 = out_shape + [jax.ShapeDtypeStruct((1, 1), jnp.float32)]
        out_specs2 = out_specs + [const((1, 1))]
        return pl.pallas_call(body, out_shape=out_shape2, grid=(NB,),
                              in_specs=in_specs, out_specs=out_specs2)(*args)


def _tc_final_body(xwp_ref, acc_ref, aself_ref, b_ref, batch_ref,
                   linW_ref, linb_ref, out_ref):
    es = jnp.exp(aself_ref[...])
    xw = xwp_ref[:, :H]
    num = es * xw + acc_ref[0, :, :H] + acc_ref[1, :, :H]
    den = es + acc_ref[0, :, H:H + 1] + acc_ref[1, :, H:H + 1]
    h = num / den + b_ref[...]
    seg = jax.lax.broadcasted_iota(jnp.int32, (B, N), 0)
    M = jnp.where(batch_ref[...] == seg, 1.0, 0.0)
    cnt = jnp.sum(M, axis=1, keepdims=True)
    pooled = jnp.dot(M, h, preferred_element_type=jnp.float32)
    pooled = pooled / jnp.maximum(cnt, 1.0)
    logit = jnp.dot(pooled, linW_ref[...], preferred_element_type=jnp.float32)
    out_ref[...] = jax.nn.sigmoid(logit + linb_ref[...])


def _sc_edge_body(xwp_hbm, adst_hbm, c16_hbm, e3_hbm,
                  acc_hbm,
                  adst_v, c_v, idxr, gb0, gb1, gb2, gb3, sb0, sb1, acc_sh,
                  sem_g0, sem_g1, sem_g2, sem_g3, sem_s0, sem_s1, isem):
    cid = lax.axis_index("c")
    sid = lax.axis_index("s")
    wid = sid * NC + cid

    pltpu.sync_copy(adst_hbm, adst_v)
    pltpu.sync_copy(c16_hbm, c_v)

    def zero_row(r, _):
        for off in list(range(0, H, 16)) + [HP - 16]:
            sb0[r, pl.ds(off, 16)] = jnp.zeros((16,), jnp.float32)
        return ()
    lax.fori_loop(0, CK, zero_row, (), unroll=4)
    base = sid * RPT
    nz = RPT // CK
    rem = RPT % CK
    for k in range(nz):
        pltpu.async_copy(sb0, acc_sh.at[pl.ds(base + k * CK, CK)], sem_s0)
    if rem:
        pltpu.async_copy(sb0.at[pl.ds(0, rem)],
                         acc_sh.at[pl.ds(base + nz * CK, rem)], sem_s0)
    for k in range(nz):
        pltpu.make_async_copy(sb0, acc_sh.at[pl.ds(base + k * CK, CK)],
                              sem_s0).wait()
    if rem:
        pltpu.make_async_copy(sb0.at[pl.ds(0, rem)],
                              acc_sh.at[pl.ds(base + nz * CK, rem)],
                              sem_s0).wait()
    plsc.subcore_barrier()

    c_vec = c_v[...]
    gbufs = (gb0, gb1, gb2, gb3)
    sbufs = (sb0, sb1)
    gsems = (sem_g0, sem_g1, sem_g2, sem_g3)
    ssems = (sem_s0, sem_s1)

    def slot_of(ch):
        return lax.rem(ch, NSLOT)

    def stage_idx(ch):
        sl = slot_of(ch)
        pltpu.async_copy(e3_hbm.at[wid, ch], idxr.at[sl], isem.at[sl])

    def wait_idx(ch):
        sl = slot_of(ch)
        pltpu.make_async_copy(e3_hbm.at[wid, ch], idxr.at[sl],
                              isem.at[sl]).wait()

    def issue_gather(ch, g):
        sl = slot_of(ch)
        pltpu.async_copy(xwp_hbm.at[idxr.at[sl, 0]], gbufs[g], gsems[g])

    def wait_gather(ch, g):
        sl = slot_of(ch)
        pltpu.make_async_copy(xwp_hbm.at[idxr.at[sl, 0]], gbufs[g],
                              gsems[g]).wait()

    def issue_scatter(ch, s):
        sl = slot_of(ch)
        pltpu.async_copy(sbufs[s], acc_sh.at[idxr.at[sl, 1]], ssems[s],
                         add=True)

    def wait_scatter(ch, s):
        sl = slot_of(ch)
        pltpu.make_async_copy(sbufs[s], acc_sh.at[idxr.at[sl, 1]],
                              ssems[s]).wait()

    def do_chunk(ch, g, s, steady=True):
        wait_gather(ch, g)
        sl = slot_of(ch)

        if steady:
            wait_scatter(ch - NSB, s)
        elif isinstance(ch, int) and ch >= NSB:
            wait_scatter(ch - NSB, s)

        col129 = jnp.full((16,), H + 1, jnp.int32)
        for off, j0 in ((0, 0), (16, 0), (24, 8)):
            rows16 = lax.iota(jnp.int32, 16) + off
            asrc_g = plsc.load_gather(gbufs[g], [rows16, col129])
            d16 = idxr[sl, 1, pl.ds(off, 16)]
            ewbits = idxr[sl, 2, pl.ds(off, 16)]
            ew16 = plsc.bitcast(ewbits, jnp.float32)
            adst_g = plsc.load_gather(adst_v, [d16])
            t = asrc_g + adst_g + c_vec * ew16
            alpha = jnp.where(t >= 0, t, 0.2 * t)
            w16 = jnp.exp(alpha)
            for j in range(j0, 16):
                r = off + j
                wspl = jnp.full((16,), w16[j])
                sbufs[s][r, pl.ds(HP - 16, 16)] = wspl
                for v in range(H // 16):
                    cs = pl.ds(v * 16, 16)
                    sbufs[s][r, cs] = gbufs[g][r, cs] * wspl

        issue_scatter(ch, s)

        if steady:
            wait_idx(ch + NGB)
            issue_gather(ch + NGB, g)
            stage_idx(ch + NGB + 2)
        else:
            if not isinstance(ch, int) or ch + NGB < NCH:
                wait_idx(ch + NGB)
                issue_gather(ch + NGB, g)
            if not isinstance(ch, int) or ch + NGB + 2 < NCH:
                stage_idx(ch + NGB + 2)

    for ch in range(NGB + 2):
        stage_idx(ch)
    for ch in range(NGB):
        wait_idx(ch)
        issue_gather(ch, ch)

    STEP = NGB
    head = NSB
    tail_start = NCH - NGB - 2
    while (tail_start - head) % STEP:
        tail_start -= 1
    for ch in range(head):
        do_chunk(ch, ch % NGB, ch % NSB, steady=False)

    def p2(i, _):
        for k in range(STEP):
            ch = head + i * STEP + k
            do_chunk(ch, (head + k) % NGB, (head + k) % NSB)
        return ()
    lax.fori_loop(0, (tail_start - head) // STEP, p2, ())
    for ch in range(tail_start, NCH):
        do_chunk(ch, ch % NGB, ch % NSB, steady=False)

    for s in range(NSB):
        ch = NCH - NSB + s
        wait_scatter(ch, ch % NSB)

    plsc.subcore_barrier()

    pltpu.sync_copy(acc_sh.at[pl.ds(base, RPT)],
                    acc_hbm.at[cid, pl.ds(base, RPT)])


def _sc_edge_call(xwp, adst, c16, e3):
    mesh = plsc.VectorSubcoreMesh(core_axis_name="c", subcore_axis_name="s",
                                  num_cores=NC, num_subcores=NS)
    f32 = jnp.float32
    kern = pl.kernel(
        _sc_edge_body,
        out_type=jax.ShapeDtypeStruct((NC, N, HP), f32),
        mesh=mesh,
        compiler_params=pltpu.CompilerParams(use_tc_tiling_on_sc=False,
                                             needs_layout_passes=False),
        scratch_types=[
            pltpu.VMEM((N,), f32),
            pltpu.VMEM((16,), f32),
            pltpu.VMEM((NSLOT, 3, CK), jnp.int32),
            pltpu.VMEM((CK, HP), f32),
            pltpu.VMEM((CK, HP), f32),
            pltpu.VMEM((CK, HP), f32),
            pltpu.VMEM((CK, HP), f32),
            pltpu.VMEM((CK, HP), f32),
            pltpu.VMEM((CK, HP), f32),
            pltpu.VMEM_SHARED((N, HP), f32),
            pltpu.SemaphoreType.DMA,
            pltpu.SemaphoreType.DMA,
            pltpu.SemaphoreType.DMA,
            pltpu.SemaphoreType.DMA,
            pltpu.SemaphoreType.DMA,
            pltpu.SemaphoreType.DMA,
            pltpu.SemaphoreType.DMA((NSLOT,)),
        ],
    )
    return kern(xwp, adst, c16, e3)


def kernel(x, edge_index, edge_weight, batch,
           W1, as1, ad1, We1, ae1, b1,
           W2, as2, ad2, We2, ae2, b2,
           W3, as3, ad3, We3, ae3, b3,
           linW, linb):
    f32 = jnp.float32
    src = edge_index[0]
    dst = edge_index[1]
    ew_bits = lax.bitcast_convert_type(edge_weight[:, 0], jnp.int32)
    e3 = jnp.stack([src.reshape(NW, NCH, CK),
                    dst.reshape(NW, NCH, CK),
                    ew_bits.reshape(NW, NCH, CK)], axis=2)

    ew2d = edge_weight.reshape(E // H, H)

    layers = [
        (W1, as1, ad1, We1, ae1, None),
        (W2, as2, ad2, We2, ae2, b1),
        (W3, as3, ad3, We3, ae3, b2),
    ]

    xwp = None
    acc = None
    aself = None
    mew = None
    for (W, a_s, a_d, We, a_e, b_prev) in layers:
        if acc is None:
            xwp, adst, aself, c16, mew = _tc_layer(
                x, None, None, None,
                W, a_s.reshape(H, 1), a_d.reshape(H, 1),
                We, a_e.reshape(1, H), ew2d)
        else:
            xwp, adst, aself, c16 = _tc_layer(
                xwp, acc, aself, b_prev.reshape(1, H),
                W, a_s.reshape(H, 1), a_d.reshape(H, 1),
                We, a_e.reshape(1, H), mew)
        acc = _sc_edge_call(xwp, adst.reshape(N), c16.reshape(16), e3)

    out = pl.pallas_call(
        _tc_final_body,
        out_shape=jax.ShapeDtypeStruct((B, 1), f32),
    )(xwp, acc, aself, b3.reshape(1, H), batch.reshape(1, N), linW,
      linb.reshape(1, 1))
    return out
